# Initial kernel scaffold; baseline (speedup 1.0000x reference)
import jax
import jax.numpy as jnp
from jax.experimental import pallas as pl


def kernel(x, edge_index, edge_weight, W0, W1, W2, Wlin):
    raise NotImplementedError("write your pallas kernel here")


# speedup vs baseline: 27.5093x; 27.5093x over previous
import functools

import jax
import jax.numpy as jnp
from jax import lax
from jax.experimental import pallas as pl
from jax.experimental.pallas import tpu as pltpu
from jax.experimental.pallas import tpu_sc as plsc

NC = 2
NS = 16
NW = NC * NS

F32 = jnp.float32
I32 = jnp.int32


def _mesh():
    return plsc.VectorSubcoreMesh(core_axis_name="c", subcore_axis_name="s")


_SC_PARAMS = pltpu.CompilerParams(needs_layout_passes=False,
                                  use_tc_tiling_on_sc=False)


def _sc_deg(n, e):
    epw = e // NW
    K = 2048
    nwin = epw // K
    npc = n // NS

    def body(row_h, col_h, ew_h, zero_h, degp_h, sp_h, cp_h,
             deg_s, s_s, c_s, rowb, colb, ewb, ub_deg, ub_s, ub_c):
        cid = lax.axis_index("c")
        sid = lax.axis_index("s")
        wid = cid * NS + sid
        sl = pl.ds(sid * npc, npc)
        pltpu.sync_copy(zero_h.at[sl], deg_s.at[sl])
        pltpu.sync_copy(zero_h.at[sl], s_s.at[sl])
        pltpu.sync_copy(zero_h.at[sl], c_s.at[sl])
        plsc.subcore_barrier()
        base0 = wid * epw
        zero16 = jnp.zeros((16,), F32)
        one16 = jnp.ones((16,), F32)

        def win(w, carry):
            base = base0 + w * K
            pltpu.sync_copy(row_h.at[pl.ds(base, K)], rowb)
            pltpu.sync_copy(col_h.at[pl.ds(base, K)], colb)
            pltpu.sync_copy(ew_h.at[pl.ds(base, K)], ewb)

            def inner(i, c2):
                ix = pl.ds(i * 16, 16)
                rv = rowb[ix]
                cv = colb[ix]
                ev = ewb[ix]
                eq = rv == cv
                ub_deg[ix] = jnp.where(eq, zero16, ev)
                ub_s[ix] = jnp.where(eq, ev, zero16)
                ub_c[ix] = jnp.where(eq, one16, zero16)
                return c2

            lax.fori_loop(0, K // 16, inner, 0)
            pltpu.sync_copy(ub_deg, deg_s.at[colb], add=True)
            pltpu.sync_copy(ub_s, s_s.at[rowb], add=True)
            pltpu.sync_copy(ub_c, c_s.at[rowb], add=True)
            return carry

        lax.fori_loop(0, nwin, win, 0)
        plsc.subcore_barrier()
        pltpu.sync_copy(deg_s.at[sl], degp_h.at[cid, sl])
        pltpu.sync_copy(s_s.at[sl], sp_h.at[cid, sl])
        pltpu.sync_copy(c_s.at[sl], cp_h.at[cid, sl])

    return pl.kernel(
        body,
        out_type=[jax.ShapeDtypeStruct((NC, n), F32)] * 3,
        mesh=_mesh(),
        compiler_params=_SC_PARAMS,
        scratch_types=[
            pltpu.VMEM_SHARED((n,), F32),
            pltpu.VMEM_SHARED((n,), F32),
            pltpu.VMEM_SHARED((n,), F32),
            pltpu.VMEM((K,), I32),
            pltpu.VMEM((K,), I32),
            pltpu.VMEM((K,), F32),
            pltpu.VMEM((K,), F32),
            pltpu.VMEM((K,), F32),
            pltpu.VMEM((K,), F32),
        ],
    )


def _sc_norm(n, e):
    epw = e // NW
    K = 2048
    nwin = epw // K

    def body(row_h, col_h, ew_h, dis_h, norm_h,
             disv, rowb, colb, ewb, nrmb):
        cid = lax.axis_index("c")
        sid = lax.axis_index("s")
        wid = cid * NS + sid
        pltpu.sync_copy(dis_h, disv)
        base0 = wid * epw
        zero16 = jnp.zeros((16,), F32)

        def win(w, carry):
            base = base0 + w * K
            pltpu.sync_copy(row_h.at[pl.ds(base, K)], rowb)
            pltpu.sync_copy(col_h.at[pl.ds(base, K)], colb)
            pltpu.sync_copy(ew_h.at[pl.ds(base, K)], ewb)

            def inner(i, c2):
                ix = pl.ds(i * 16, 16)
                rv = rowb[ix]
                cv = colb[ix]
                ev = ewb[ix]
                dr = plsc.load_gather(disv, [rv])
                dc = plsc.load_gather(disv, [cv])
                ew0 = jnp.where(rv == cv, zero16, ev)
                nrmb[ix] = dr * ew0 * dc
                return c2

            lax.fori_loop(0, K // 16, inner, 0)
            pltpu.sync_copy(nrmb, norm_h.at[pl.ds(base, K)])
            return carry

        lax.fori_loop(0, nwin, win, 0)

    return pl.kernel(
        body,
        out_type=jax.ShapeDtypeStruct((e,), F32),
        mesh=_mesh(),
        compiler_params=_SC_PARAMS,
        scratch_types=[
            pltpu.VMEM((n,), F32),
            pltpu.VMEM((K,), I32),
            pltpu.VMEM((K,), I32),
            pltpu.VMEM((K,), F32),
            pltpu.VMEM((K,), F32),
        ],
    )


def _sc_layer(n, e, hd):
    epw = e // NW
    K = 1024
    nwin = epw // K
    npc = n // NS

    def body(h_h, row_h, col_h, nrm_h, zero_h, out_h,
             h_s, o_s, rowb, colb, nrmb, rows2, upd2):
        cid = lax.axis_index("c")
        sid = lax.axis_index("s")
        wid = cid * NS + sid
        sl = pl.ds(sid * npc, npc)
        pltpu.sync_copy(h_h.at[sl], h_s.at[sl])
        pltpu.sync_copy(zero_h.at[sl], o_s.at[sl])
        plsc.subcore_barrier()
        base0 = wid * epw
        iota = lax.iota(I32, 16)
        lane8 = lax.shift_right_logical(iota, 3)
        lane7 = lax.bitwise_and(iota, jnp.full((16,), 7, I32))

        def win(w, carry):
            base = base0 + w * K
            pltpu.sync_copy(row_h.at[pl.ds(base, K)], rowb)
            pltpu.sync_copy(col_h.at[pl.ds(base, K)], colb)
            pltpu.sync_copy(nrm_h.at[pl.ds(base, K)], nrmb)
            pltpu.sync_copy(h_s.at[rowb], rows2)

            def inner(i, c2):
                e_idx = lane8 + i * 2
                v = plsc.load_gather(rows2, [e_idx, lane7])
                nv = plsc.load_gather(nrmb, [e_idx])
                plsc.store_scatter(upd2, [e_idx, lane7], v * nv)
                return c2

            lax.fori_loop(0, K // 2, inner, 0)
            pltpu.sync_copy(upd2, o_s.at[colb], add=True)
            return carry

        lax.fori_loop(0, nwin, win, 0)
        plsc.subcore_barrier()
        pltpu.sync_copy(o_s.at[sl], out_h.at[cid, sl])

    return pl.kernel(
        body,
        out_type=jax.ShapeDtypeStruct((NC, n, hd), F32),
        mesh=_mesh(),
        compiler_params=_SC_PARAMS,
        scratch_types=[
            pltpu.VMEM_SHARED((n, hd), F32),
            pltpu.VMEM_SHARED((n, hd), F32),
            pltpu.VMEM((K,), I32),
            pltpu.VMEM((K,), I32),
            pltpu.VMEM((K,), F32),
            pltpu.VMEM((K, hd), F32),
            pltpu.VMEM((K, hd), F32),
        ],
    )


def _tc_matmul(n, in_dim, hd):
    B = 1024

    def body(x_ref, w_ref, o_ref):
        o_ref[...] = jnp.dot(x_ref[...], w_ref[...],
                             preferred_element_type=F32)

    return pl.pallas_call(
        body,
        grid=(n // B,),
        in_specs=[pl.BlockSpec((B, in_dim), lambda i: (i, 0)),
                  pl.BlockSpec((in_dim, hd), lambda i: (0, 0))],
        out_specs=pl.BlockSpec((B, hd), lambda i: (i, 0)),
        out_shape=jax.ShapeDtypeStruct((n, hd), F32),
    )


def _tc_prep(n):
    r = n // 128

    def body(degp, sp, cp, dis_o, d_o):
        raw = degp[0] + degp[1]
        s = sp[0] + sp[1]
        c = cp[0] + cp[1]
        loop_w = jnp.where(c > 0, s / jnp.maximum(c, 1.0),
                           jnp.ones_like(c))
        deg = raw + loop_w
        dis = jnp.where(deg > 0, lax.rsqrt(jnp.where(deg > 0, deg, 1.0)),
                        jnp.zeros_like(deg))
        dis_o[...] = dis
        d_o[...] = dis * dis * loop_w

    return pl.pallas_call(
        body,
        out_shape=[jax.ShapeDtypeStruct((r, 128), F32)] * 2,
    )


def _tc_comb(n, hd, w1_dim, w2_dim):
    B = 4096

    def body(*refs):
        if w2_dim:
            p_ref, d_ref, h_ref, w_ref, w2_ref, o_ref = refs
        elif w1_dim:
            p_ref, d_ref, h_ref, w_ref, o_ref = refs
        else:
            p_ref, d_ref, h_ref, o_ref = refs
        agg = p_ref[0] + p_ref[1] + d_ref[...] * h_ref[...]
        if w1_dim:
            agg = jnp.dot(agg, w_ref[...], preferred_element_type=F32)
        z = jnp.maximum(agg, 0.0)
        if w2_dim:
            z = jnp.dot(z, w2_ref[...], preferred_element_type=F32)
        o_ref[...] = z

    in_specs = [pl.BlockSpec((NC, B, hd), lambda i: (0, i, 0)),
                pl.BlockSpec((B, 1), lambda i: (i, 0)),
                pl.BlockSpec((B, hd), lambda i: (i, 0))]
    out_dim = hd
    if w1_dim:
        in_specs.append(pl.BlockSpec((hd, w1_dim), lambda i: (0, 0)))
        out_dim = w1_dim
    if w2_dim:
        in_specs.append(pl.BlockSpec((w1_dim, w2_dim), lambda i: (0, 0)))
        out_dim = w2_dim

    return pl.pallas_call(
        body,
        grid=(n // B,),
        in_specs=in_specs,
        out_specs=pl.BlockSpec((B, out_dim), lambda i: (i, 0)),
        out_shape=jax.ShapeDtypeStruct((n, out_dim), F32),
    )


def kernel(x, edge_index, edge_weight, W0, W1, W2, Wlin):
    n, in_dim = x.shape
    e = edge_index.shape[1]
    hd = W0.shape[1]
    row = edge_index[0]
    col = edge_index[1]
    zeros1 = jnp.zeros((n,), F32)
    zeros2 = jnp.zeros((n, hd), F32)

    degp, sp, cp = _sc_deg(n, e)(row, col, edge_weight, zeros1)
    r = n // 128
    dis2d, d2d = _tc_prep(n)(degp.reshape(NC, r, 128),
                             sp.reshape(NC, r, 128),
                             cp.reshape(NC, r, 128))
    dis = dis2d.reshape(n)
    d = d2d.reshape(n, 1)

    nrm = _sc_norm(n, e)(row, col, edge_weight, dis)
    h0 = _tc_matmul(n, in_dim, hd)(x, W0)

    layer = _sc_layer(n, e, hd)
    p = layer(h0, row, col, nrm, zeros2)
    h1 = _tc_comb(n, hd, 0, 0)(p, d, h0)
    p = layer(h1, row, col, nrm, zeros2)
    h2 = _tc_comb(n, hd, hd, 0)(p, d, h1, W1)
    p = layer(h2, row, col, nrm, zeros2)
    pred_full = _tc_comb(n, hd, hd, Wlin.shape[1])(p, d, h2, W2, Wlin)

    g = 6
    return pred_full.reshape(n // g, g, Wlin.shape[1])[:, 0, :]


# speedup vs baseline: 56.2477x; 2.0447x over previous
import functools

import jax
import jax.numpy as jnp
from jax import lax
from jax.experimental import pallas as pl
from jax.experimental.pallas import tpu as pltpu
from jax.experimental.pallas import tpu_sc as plsc

NC = 2
NS = 16
NW = NC * NS

F32 = jnp.float32
I32 = jnp.int32


def _mesh():
    return plsc.VectorSubcoreMesh(core_axis_name="c", subcore_axis_name="s")


_SC_PARAMS = pltpu.CompilerParams(needs_layout_passes=False,
                                  use_tc_tiling_on_sc=False)


def _sc_deg(n, e):
    epw = e // NW
    K = 2048
    nwin = epw // K
    npc = n // NS

    def body(row_h, col_h, ew_h, zero_h, degp_h, sp_h, cp_h,
             deg_s, s_s, c_s, rowb, colb, ewb, ub_deg, ub_s, ub_c):
        cid = lax.axis_index("c")
        sid = lax.axis_index("s")
        wid = cid * NS + sid
        sl = pl.ds(sid * npc, npc)
        pltpu.sync_copy(zero_h.at[sl], deg_s.at[sl])
        pltpu.sync_copy(zero_h.at[sl], s_s.at[sl])
        pltpu.sync_copy(zero_h.at[sl], c_s.at[sl])
        plsc.subcore_barrier()
        base0 = wid * epw
        zero16 = jnp.zeros((16,), F32)
        one16 = jnp.ones((16,), F32)

        def win(w, carry):
            base = base0 + w * K
            pltpu.sync_copy(row_h.at[pl.ds(base, K)], rowb)
            pltpu.sync_copy(col_h.at[pl.ds(base, K)], colb)
            pltpu.sync_copy(ew_h.at[pl.ds(base, K)], ewb)

            def inner(i, c2):
                ix = pl.ds(i * 16, 16)
                rv = rowb[ix]
                cv = colb[ix]
                ev = ewb[ix]
                eq = rv == cv
                ub_deg[ix] = jnp.where(eq, zero16, ev)
                ub_s[ix] = jnp.where(eq, ev, zero16)
                ub_c[ix] = jnp.where(eq, one16, zero16)
                return c2

            lax.fori_loop(0, K // 16, inner, 0)
            pltpu.sync_copy(ub_deg, deg_s.at[colb], add=True)
            pltpu.sync_copy(ub_s, s_s.at[rowb], add=True)
            pltpu.sync_copy(ub_c, c_s.at[rowb], add=True)
            return carry

        lax.fori_loop(0, nwin, win, 0)
        plsc.subcore_barrier()
        pltpu.sync_copy(deg_s.at[sl], degp_h.at[cid, sl])
        pltpu.sync_copy(s_s.at[sl], sp_h.at[cid, sl])
        pltpu.sync_copy(c_s.at[sl], cp_h.at[cid, sl])

    return pl.kernel(
        body,
        out_type=[jax.ShapeDtypeStruct((NC, n), F32)] * 3,
        mesh=_mesh(),
        compiler_params=_SC_PARAMS,
        scratch_types=[
            pltpu.VMEM_SHARED((n,), F32),
            pltpu.VMEM_SHARED((n,), F32),
            pltpu.VMEM_SHARED((n,), F32),
            pltpu.VMEM((K,), I32),
            pltpu.VMEM((K,), I32),
            pltpu.VMEM((K,), F32),
            pltpu.VMEM((K,), F32),
            pltpu.VMEM((K,), F32),
            pltpu.VMEM((K,), F32),
        ],
    )


def _sc_norm(n, e):
    epw = e // NW
    K = 2048
    nwin = epw // K

    def body(row_h, col_h, ew_h, dis_h, norm_h,
             disv, rowb, colb, ewb, nrmb):
        cid = lax.axis_index("c")
        sid = lax.axis_index("s")
        wid = cid * NS + sid
        pltpu.sync_copy(dis_h, disv)
        base0 = wid * epw
        zero16 = jnp.zeros((16,), F32)

        def win(w, carry):
            base = base0 + w * K
            pltpu.sync_copy(row_h.at[pl.ds(base, K)], rowb)
            pltpu.sync_copy(col_h.at[pl.ds(base, K)], colb)
            pltpu.sync_copy(ew_h.at[pl.ds(base, K)], ewb)

            def inner(i, c2):
                ix = pl.ds(i * 16, 16)
                rv = rowb[ix]
                cv = colb[ix]
                ev = ewb[ix]
                dr = plsc.load_gather(disv, [rv])
                dc = plsc.load_gather(disv, [cv])
                ew0 = jnp.where(rv == cv, zero16, ev)
                nrmb[ix] = dr * ew0 * dc
                return c2

            lax.fori_loop(0, K // 16, inner, 0)
            pltpu.sync_copy(nrmb, norm_h.at[pl.ds(base, K)])
            return carry

        lax.fori_loop(0, nwin, win, 0)

    return pl.kernel(
        body,
        out_type=jax.ShapeDtypeStruct((e,), F32),
        mesh=_mesh(),
        compiler_params=_SC_PARAMS,
        scratch_types=[
            pltpu.VMEM((n,), F32),
            pltpu.VMEM((K,), I32),
            pltpu.VMEM((K,), I32),
            pltpu.VMEM((K,), F32),
            pltpu.VMEM((K,), F32),
        ],
    )


def _sc_layer(n, e, hd):
    epw = e // NW
    K = 1024
    nwin = epw // K
    npc = n // NS

    def body(h_h, row_h, col_h, nrm_h, zero_h, out_h,
             h_s, o_s, rowb, colb, nrmb, rows2, upd2):
        cid = lax.axis_index("c")
        sid = lax.axis_index("s")
        wid = cid * NS + sid
        sl = pl.ds(sid * npc, npc)
        pltpu.sync_copy(h_h.at[sl], h_s.at[sl])
        pltpu.sync_copy(zero_h.at[sl], o_s.at[sl])
        plsc.subcore_barrier()
        base0 = wid * epw
        iota = lax.iota(I32, 16)
        lane8 = lax.shift_right_logical(iota, 3)
        lane7 = lax.bitwise_and(iota, jnp.full((16,), 7, I32))

        def win(w, carry):
            base = base0 + w * K
            pltpu.sync_copy(row_h.at[pl.ds(base, K)], rowb)
            pltpu.sync_copy(col_h.at[pl.ds(base, K)], colb)
            pltpu.sync_copy(nrm_h.at[pl.ds(base, K)], nrmb)
            pltpu.sync_copy(h_s.at[rowb], rows2)

            def inner(i, c2):
                e_idx = lane8 + i * 2
                v = plsc.load_gather(rows2, [e_idx, lane7])
                nv = plsc.load_gather(nrmb, [e_idx])
                plsc.store_scatter(upd2, [e_idx, lane7], v * nv)
                return c2

            lax.fori_loop(0, K // 2, inner, 0)
            pltpu.sync_copy(upd2, o_s.at[colb], add=True)
            return carry

        lax.fori_loop(0, nwin, win, 0)
        plsc.subcore_barrier()
        pltpu.sync_copy(o_s.at[sl], out_h.at[cid, sl])

    return pl.kernel(
        body,
        out_type=jax.ShapeDtypeStruct((NC, n, hd), F32),
        mesh=_mesh(),
        compiler_params=_SC_PARAMS,
        scratch_types=[
            pltpu.VMEM_SHARED((n, hd), F32),
            pltpu.VMEM_SHARED((n, hd), F32),
            pltpu.VMEM((K,), I32),
            pltpu.VMEM((K,), I32),
            pltpu.VMEM((K,), F32),
            pltpu.VMEM((K, hd), F32),
            pltpu.VMEM((K, hd), F32),
        ],
    )


def _tc_matmul(n, in_dim, hd):
    P = 128 // hd
    Bp = 512

    def body(x_ref, m_ref, o_ref):
        acc = jnp.zeros((Bp, 128), F32)
        for v in range(P):
            acc = acc + jnp.dot(x_ref[:, v, :], m_ref[v],
                                preferred_element_type=F32)
        o_ref[...] = acc

    return pl.pallas_call(
        body,
        grid=(n // P // Bp,),
        in_specs=[pl.BlockSpec((Bp, P, in_dim), lambda i: (i, 0, 0)),
                  pl.BlockSpec((P, in_dim, 128), lambda i: (0, 0, 0))],
        out_specs=pl.BlockSpec((Bp, 128), lambda i: (i, 0)),
        out_shape=jax.ShapeDtypeStruct((n // P, 128), F32),
    )


def _tc_prep(n, hd):
    r = n // 128

    def body(degp, sp, cp, dis_o, d_o):
        raw = degp[0] + degp[1]
        s = sp[0] + sp[1]
        c = cp[0] + cp[1]
        loop_w = jnp.where(c > 0, s / jnp.maximum(c, 1.0),
                           jnp.ones_like(c))
        deg = raw + loop_w
        dis = jnp.where(deg > 0, lax.rsqrt(jnp.where(deg > 0, deg, 1.0)),
                        jnp.zeros_like(deg))
        dis_o[...] = dis
        d_o[...] = dis * dis * loop_w

    return pl.pallas_call(
        body,
        out_shape=[jax.ShapeDtypeStruct((r, 128), F32),
                   jax.ShapeDtypeStruct((r, 128), F32)],
    )


def _tc_comb(n, hd, with_w, w2_dim):
    P = 128 // hd
    npk = n // P
    Bp = 512

    def body(*refs):
        if w2_dim:
            p_ref, d_ref, h_ref, w_ref, w2_ref, o_ref = refs
        elif with_w:
            p_ref, d_ref, h_ref, w_ref, o_ref = refs
        else:
            p_ref, d_ref, h_ref, o_ref = refs
        agg = p_ref[0] + p_ref[1] + d_ref[...] * h_ref[...]
        if with_w:
            agg = jnp.dot(agg, w_ref[...], preferred_element_type=F32)
        z = jnp.maximum(agg, 0.0)
        if w2_dim:
            z = jnp.dot(z, w2_ref[...], preferred_element_type=F32)
        o_ref[...] = z

    in_specs = [pl.BlockSpec((NC, Bp, 128), lambda i: (0, i, 0)),
                pl.BlockSpec((Bp, 128), lambda i: (i, 0)),
                pl.BlockSpec((Bp, 128), lambda i: (i, 0))]
    if with_w:
        in_specs.append(pl.BlockSpec((128, 128), lambda i: (0, 0)))
    if w2_dim:
        odim = P * w2_dim
        in_specs.append(pl.BlockSpec((128, odim), lambda i: (0, 0)))
        out_spec = pl.BlockSpec((Bp, odim), lambda i: (i, 0))
        out_shape = jax.ShapeDtypeStruct((npk, odim), F32)
    else:
        out_spec = pl.BlockSpec((Bp, 128), lambda i: (i, 0))
        out_shape = jax.ShapeDtypeStruct((npk, 128), F32)

    return pl.pallas_call(
        body,
        grid=(npk // Bp,),
        in_specs=in_specs,
        out_specs=out_spec,
        out_shape=out_shape,
    )


def kernel(x, edge_index, edge_weight, W0, W1, W2, Wlin):
    n, in_dim = x.shape
    e = edge_index.shape[1]
    hd = W0.shape[1]
    row = edge_index[0]
    col = edge_index[1]
    zeros1 = jnp.zeros((n,), F32)
    zeros2 = jnp.zeros((n, hd), F32)

    P = 128 // hd
    npk = n // P
    r = n // 128

    degp, sp, cp = _sc_deg(n, e)(row, col, edge_weight, zeros1)
    dis2d, d2d = _tc_prep(n, hd)(degp.reshape(NC, r, 128),
                                 sp.reshape(NC, r, 128),
                                 cp.reshape(NC, r, 128))
    dis = dis2d.reshape(n)
    d_pk = jnp.repeat(d2d.reshape(n), hd).reshape(npk, 128)

    nrm = _sc_norm(n, e)(row, col, edge_weight, dis)
    m0 = jnp.stack([jnp.pad(W0, ((0, 0), (v * hd, 128 - (v + 1) * hd)))
                    for v in range(P)])
    h0_pk = _tc_matmul(n, in_dim, hd)(x.reshape(npk, P, in_dim), m0)

    eye_p = jnp.eye(P, dtype=F32)
    w1bd = jnp.kron(eye_p, W1)
    w2bd = jnp.kron(eye_p, W2)
    wlbd = jnp.kron(eye_p, Wlin)

    layer = _sc_layer(n, e, hd)
    comb_relu = _tc_comb(n, hd, False, 0)
    comb_w = _tc_comb(n, hd, True, 0)

    p = layer(h0_pk.reshape(n, hd), row, col, nrm, zeros2)
    h1_pk = comb_relu(p.reshape(NC, npk, 128), d_pk, h0_pk)
    p = layer(h1_pk.reshape(n, hd), row, col, nrm, zeros2)
    h2_pk = comb_w(p.reshape(NC, npk, 128), d_pk, h1_pk, w1bd)
    p = layer(h2_pk.reshape(n, hd), row, col, nrm, zeros2)
    pred_pk = _tc_comb(n, hd, True, Wlin.shape[1])(p.reshape(NC, npk, 128),
                                                   d_pk, h2_pk, w2bd, wlbd)

    g = 6
    return pred_pk.reshape(n)[::g].reshape(n // g, 1)


# speedup vs baseline: 71.2188x; 1.2662x over previous
import functools

import jax
import jax.numpy as jnp
from jax import lax
from jax.experimental import pallas as pl
from jax.experimental.pallas import tpu as pltpu
from jax.experimental.pallas import tpu_sc as plsc

NC = 2
NS = 16
NW = NC * NS

F32 = jnp.float32
I32 = jnp.int32


def _mesh():
    return plsc.VectorSubcoreMesh(core_axis_name="c", subcore_axis_name="s")


_SC_PARAMS = pltpu.CompilerParams(needs_layout_passes=False,
                                  use_tc_tiling_on_sc=False)


def _sc_deg(n, e):
    epw = e // NW
    K = 2048
    nwin = epw // K
    npc = n // NS

    def body(row_h, col_h, ew_h, zero_h, degp_h, sp_h, cp_h,
             deg_s, s_s, c_s, rowb, colb, ewb, ub_deg, ub_s, ub_c):
        cid = lax.axis_index("c")
        sid = lax.axis_index("s")
        wid = cid * NS + sid
        sl = pl.ds(sid * npc, npc)
        pltpu.sync_copy(zero_h.at[sl], deg_s.at[sl])
        pltpu.sync_copy(zero_h.at[sl], s_s.at[sl])
        pltpu.sync_copy(zero_h.at[sl], c_s.at[sl])
        plsc.subcore_barrier()
        base0 = wid * epw
        zero16 = jnp.zeros((16,), F32)
        one16 = jnp.ones((16,), F32)

        def win(w, carry):
            base = base0 + w * K
            pltpu.sync_copy(row_h.at[pl.ds(base, K)], rowb)
            pltpu.sync_copy(col_h.at[pl.ds(base, K)], colb)
            pltpu.sync_copy(ew_h.at[pl.ds(base, K)], ewb)

            def inner(i, c2):
                ix = pl.ds(i * 16, 16)
                rv = rowb[ix]
                cv = colb[ix]
                ev = ewb[ix]
                eq = rv == cv
                ub_deg[ix] = jnp.where(eq, zero16, ev)
                ub_s[ix] = jnp.where(eq, ev, zero16)
                ub_c[ix] = jnp.where(eq, one16, zero16)
                return c2

            lax.fori_loop(0, K // 16, inner, 0)
            pltpu.sync_copy(ub_deg, deg_s.at[colb], add=True)
            pltpu.sync_copy(ub_s, s_s.at[rowb], add=True)
            pltpu.sync_copy(ub_c, c_s.at[rowb], add=True)
            return carry

        lax.fori_loop(0, nwin, win, 0)
        plsc.subcore_barrier()
        pltpu.sync_copy(deg_s.at[sl], degp_h.at[cid, sl])
        pltpu.sync_copy(s_s.at[sl], sp_h.at[cid, sl])
        pltpu.sync_copy(c_s.at[sl], cp_h.at[cid, sl])

    return pl.kernel(
        body,
        out_type=[jax.ShapeDtypeStruct((NC, n), F32)] * 3,
        mesh=_mesh(),
        compiler_params=_SC_PARAMS,
        scratch_types=[
            pltpu.VMEM_SHARED((n,), F32),
            pltpu.VMEM_SHARED((n,), F32),
            pltpu.VMEM_SHARED((n,), F32),
            pltpu.VMEM((K,), I32),
            pltpu.VMEM((K,), I32),
            pltpu.VMEM((K,), F32),
            pltpu.VMEM((K,), F32),
            pltpu.VMEM((K,), F32),
            pltpu.VMEM((K,), F32),
        ],
    )


def _sc_norm(n, e):
    epw = e // NW
    K = 2048
    nwin = epw // K

    def body(row_h, col_h, ew_h, dis_h, norm_h,
             disv, rowb, colb, ewb, nrmb):
        cid = lax.axis_index("c")
        sid = lax.axis_index("s")
        wid = cid * NS + sid
        pltpu.sync_copy(dis_h, disv)
        base0 = wid * epw
        zero16 = jnp.zeros((16,), F32)

        def win(w, carry):
            base = base0 + w * K
            pltpu.sync_copy(row_h.at[pl.ds(base, K)], rowb)
            pltpu.sync_copy(col_h.at[pl.ds(base, K)], colb)
            pltpu.sync_copy(ew_h.at[pl.ds(base, K)], ewb)

            def inner(i, c2):
                ix = pl.ds(i * 16, 16)
                rv = rowb[ix]
                cv = colb[ix]
                ev = ewb[ix]
                dr = plsc.load_gather(disv, [rv])
                dc = plsc.load_gather(disv, [cv])
                ew0 = jnp.where(rv == cv, zero16, ev)
                nrmb[ix] = dr * ew0 * dc
                return c2

            lax.fori_loop(0, K // 16, inner, 0)
            pltpu.sync_copy(nrmb, norm_h.at[pl.ds(base, K)])
            return carry

        lax.fori_loop(0, nwin, win, 0)

    return pl.kernel(
        body,
        out_type=jax.ShapeDtypeStruct((e,), F32),
        mesh=_mesh(),
        compiler_params=_SC_PARAMS,
        scratch_types=[
            pltpu.VMEM((n,), F32),
            pltpu.VMEM((K,), I32),
            pltpu.VMEM((K,), I32),
            pltpu.VMEM((K,), F32),
            pltpu.VMEM((K,), F32),
        ],
    )


def _sc_layer(n, e, hd):
    epw = e // NW
    K = 1024
    nwin = epw // K
    nout = (nwin + 3) // 4
    npc = n // NS

    def body(h_h, row_h, col_h, nrm_h, zero_h, out_h,
             h_s, o_s,
             rb0, rb1, rb2, rb3, cb0, cb1, cb2, cb3,
             nb0, nb1, nb2, nb3, r2a, r2b,
             si0, si1, si2, si3, sg0, sg1, ss0, ss1):
        rbs = [rb0, rb1, rb2, rb3]
        cbs = [cb0, cb1, cb2, cb3]
        nbs = [nb0, nb1, nb2, nb3]
        rows2 = [r2a, r2b]
        upd2 = rows2
        sis = [si0, si1, si2, si3]
        sgs = [sg0, sg1]
        sss = [ss0, ss1]

        cid = lax.axis_index("c")
        sid = lax.axis_index("s")
        wid = cid * NS + sid
        sl = pl.ds(sid * npc, npc)
        base0 = wid * epw
        iota = lax.iota(I32, 16)
        lane8 = lax.shift_right_logical(iota, 3)
        lane7 = lax.bitwise_and(iota, jnp.full((16,), 7, I32))

        def issue_streams(w, s4):
            base = base0 + w * K
            pltpu.async_copy(row_h.at[pl.ds(base, K)], rbs[s4], sis[s4])
            pltpu.async_copy(col_h.at[pl.ds(base, K)], cbs[s4], sis[s4])
            pltpu.async_copy(nrm_h.at[pl.ds(base, K)], nbs[s4], sis[s4])

        def wait_streams(s4):
            pltpu.make_async_copy(row_h.at[pl.ds(0, K)], rbs[s4], sis[s4]).wait()
            pltpu.make_async_copy(col_h.at[pl.ds(0, K)], cbs[s4], sis[s4]).wait()
            pltpu.make_async_copy(nrm_h.at[pl.ds(0, K)], nbs[s4], sis[s4]).wait()

        def issue_gather(s4, s2):
            pltpu.async_copy(h_s.at[rbs[s4]], rows2[s2], sgs[s2])

        def wait_gather(s4, s2):
            pltpu.make_async_copy(h_s.at[rbs[s4]], rows2[s2], sgs[s2]).wait()

        def issue_scatter(s4, s2):
            pltpu.async_copy(upd2[s2], o_s.at[cbs[s4]], sss[s2], add=True)

        def wait_scatter(s4, s2):
            pltpu.make_async_copy(upd2[s2], o_s.at[cbs[s4]], sss[s2]).wait()

        issue_streams(0, 0)
        issue_streams(1, 1)
        pltpu.sync_copy(h_h.at[sl], h_s.at[sl])
        pltpu.sync_copy(zero_h.at[sl], o_s.at[sl])
        plsc.subcore_barrier()
        wait_streams(0)
        issue_gather(0, 0)

        def outer(it, carry):
            w0 = it * 4
            for j in range(4):
                w = w0 + j
                s2 = j % 2

                @pl.when(w < nwin)
                def _window():
                    wait_gather(j, s2)

                    @pl.when(w + 2 < nwin)
                    def _():
                        issue_streams(w + 2, (j + 2) % 4)

                    @pl.when(w + 1 < nwin)
                    def _():
                        wait_streams((j + 1) % 4)

                        @pl.when(w >= 1)
                        def _():
                            wait_scatter((j + 3) % 4, (j + 1) % 2)

                        issue_gather((j + 1) % 4, (j + 1) % 2)

                    def inner(i, c2):
                        e_idx = lane8 + i * 2
                        v = plsc.load_gather(rows2[s2], [e_idx, lane7])
                        nv = plsc.load_gather(nbs[j], [e_idx])
                        plsc.store_scatter(upd2[s2], [e_idx, lane7], v * nv)
                        return c2

                    lax.fori_loop(0, K // 2, inner, 0)
                    issue_scatter(j, s2)

            return carry

        lax.fori_loop(0, nout, outer, 0)
        wait_scatter((nwin - 2) % 4, (nwin - 2) % 2)
        wait_scatter((nwin - 1) % 4, (nwin - 1) % 2)
        plsc.subcore_barrier()
        pltpu.sync_copy(o_s.at[sl], out_h.at[cid, sl])

    return pl.kernel(
        body,
        out_type=jax.ShapeDtypeStruct((NC, n, hd), F32),
        mesh=_mesh(),
        compiler_params=_SC_PARAMS,
        scratch_types=(
            [pltpu.VMEM_SHARED((n, hd), F32),
             pltpu.VMEM_SHARED((n, hd), F32)]
            + [pltpu.VMEM((K,), I32)] * 8
            + [pltpu.VMEM((K,), F32)] * 4
            + [pltpu.VMEM((K, hd), F32)] * 2
            + [pltpu.SemaphoreType.DMA] * 8
        ),
    )


def _tc_matmul(n, in_dim, hd):
    P = 128 // hd
    Bp = 512

    def body(x_ref, m_ref, o_ref):
        acc = jnp.zeros((Bp, 128), F32)
        for v in range(P):
            acc = acc + jnp.dot(x_ref[:, v, :], m_ref[v],
                                preferred_element_type=F32)
        o_ref[...] = acc

    return pl.pallas_call(
        body,
        grid=(n // P // Bp,),
        in_specs=[pl.BlockSpec((Bp, P, in_dim), lambda i: (i, 0, 0)),
                  pl.BlockSpec((P, in_dim, 128), lambda i: (0, 0, 0))],
        out_specs=pl.BlockSpec((Bp, 128), lambda i: (i, 0)),
        out_shape=jax.ShapeDtypeStruct((n // P, 128), F32),
    )


def _tc_prep(n, hd):
    r = n // 128

    def body(degp, sp, cp, dis_o, d_o):
        raw = degp[0] + degp[1]
        s = sp[0] + sp[1]
        c = cp[0] + cp[1]
        loop_w = jnp.where(c > 0, s / jnp.maximum(c, 1.0),
                           jnp.ones_like(c))
        deg = raw + loop_w
        dis = jnp.where(deg > 0, lax.rsqrt(jnp.where(deg > 0, deg, 1.0)),
                        jnp.zeros_like(deg))
        dis_o[...] = dis
        d_o[...] = dis * dis * loop_w

    return pl.pallas_call(
        body,
        out_shape=[jax.ShapeDtypeStruct((r, 128), F32),
                   jax.ShapeDtypeStruct((r, 128), F32)],
    )


def _tc_comb(n, hd, with_w, w2_dim):
    P = 128 // hd
    npk = n // P
    Bp = 512

    def body(*refs):
        if w2_dim:
            p_ref, d_ref, h_ref, w_ref, w2_ref, o_ref = refs
        elif with_w:
            p_ref, d_ref, h_ref, w_ref, o_ref = refs
        else:
            p_ref, d_ref, h_ref, o_ref = refs
        agg = p_ref[0] + p_ref[1] + d_ref[...] * h_ref[...]
        if with_w:
            agg = jnp.dot(agg, w_ref[...], preferred_element_type=F32)
        z = jnp.maximum(agg, 0.0)
        if w2_dim:
            z = jnp.dot(z, w2_ref[...], preferred_element_type=F32)
        o_ref[...] = z

    in_specs = [pl.BlockSpec((NC, Bp, 128), lambda i: (0, i, 0)),
                pl.BlockSpec((Bp, 128), lambda i: (i, 0)),
                pl.BlockSpec((Bp, 128), lambda i: (i, 0))]
    if with_w:
        in_specs.append(pl.BlockSpec((128, 128), lambda i: (0, 0)))
    if w2_dim:
        odim = P * w2_dim
        in_specs.append(pl.BlockSpec((128, odim), lambda i: (0, 0)))
        out_spec = pl.BlockSpec((Bp, odim), lambda i: (i, 0))
        out_shape = jax.ShapeDtypeStruct((npk, odim), F32)
    else:
        out_spec = pl.BlockSpec((Bp, 128), lambda i: (i, 0))
        out_shape = jax.ShapeDtypeStruct((npk, 128), F32)

    return pl.pallas_call(
        body,
        grid=(npk // Bp,),
        in_specs=in_specs,
        out_specs=out_spec,
        out_shape=out_shape,
    )


def kernel(x, edge_index, edge_weight, W0, W1, W2, Wlin):
    n, in_dim = x.shape
    e = edge_index.shape[1]
    hd = W0.shape[1]
    row = edge_index[0]
    col = edge_index[1]
    zeros1 = jnp.zeros((n,), F32)
    zeros2 = jnp.zeros((n, hd), F32)

    P = 128 // hd
    npk = n // P
    r = n // 128

    m0 = jnp.stack([jnp.pad(W0, ((0, 0), (v * hd, 128 - (v + 1) * hd)))
                    for v in range(P)])
    h0_pk = _tc_matmul(n, in_dim, hd)(x.reshape(npk, P, in_dim), m0)

    degp, sp, cp = _sc_deg(n, e)(row, col, edge_weight, zeros1)
    dis2d, d2d = _tc_prep(n, hd)(degp.reshape(NC, r, 128),
                                 sp.reshape(NC, r, 128),
                                 cp.reshape(NC, r, 128))
    dis = dis2d.reshape(n)
    d_pk = jnp.repeat(d2d.reshape(n), hd).reshape(npk, 128)

    nrm = _sc_norm(n, e)(row, col, edge_weight, dis)

    eye_p = jnp.eye(P, dtype=F32)
    w1bd = jnp.kron(eye_p, W1)
    w2bd = jnp.kron(eye_p, W2)
    wlbd = jnp.kron(eye_p, Wlin)

    layer = _sc_layer(n, e, hd)
    comb_relu = _tc_comb(n, hd, False, 0)
    comb_w = _tc_comb(n, hd, True, 0)

    p = layer(h0_pk.reshape(n, hd), row, col, nrm, zeros2)
    h1_pk = comb_relu(p.reshape(NC, npk, 128), d_pk, h0_pk)
    p = layer(h1_pk.reshape(n, hd), row, col, nrm, zeros2)
    h2_pk = comb_w(p.reshape(NC, npk, 128), d_pk, h1_pk, w1bd)
    p = layer(h2_pk.reshape(n, hd), row, col, nrm, zeros2)
    pred_pk = _tc_comb(n, hd, True, Wlin.shape[1])(p.reshape(NC, npk, 128),
                                                   d_pk, h2_pk, w2bd, wlbd)

    g = 6
    return pred_pk.reshape(n)[::g].reshape(n // g, 1)


# speedup vs baseline: 109.3393x; 1.5353x over previous
import functools

import jax
import jax.numpy as jnp
from jax import lax
from jax.experimental import pallas as pl
from jax.experimental.pallas import tpu as pltpu
from jax.experimental.pallas import tpu_sc as plsc

NC = 2
NS = 16
NW = NC * NS

F32 = jnp.float32
I32 = jnp.int32


def _mesh():
    return plsc.VectorSubcoreMesh(core_axis_name="c", subcore_axis_name="s")


_SC_PARAMS = pltpu.CompilerParams(needs_layout_passes=False,
                                  use_tc_tiling_on_sc=False)


def _sc_deg(n, e):
    epw = e // NW
    K = 2048
    nwin = epw // K
    npc = n // NS

    def body(row_h, col_h, ew_h, zero_h, degp_h, sp_h, cp_h,
             deg_s, s_s, c_s, rowb, colb, ewb, ub_deg, ub_s, ub_c):
        cid = lax.axis_index("c")
        sid = lax.axis_index("s")
        wid = cid * NS + sid
        sl = pl.ds(sid * npc, npc)
        pltpu.sync_copy(zero_h.at[sl], deg_s.at[sl])
        pltpu.sync_copy(zero_h.at[sl], s_s.at[sl])
        pltpu.sync_copy(zero_h.at[sl], c_s.at[sl])
        plsc.subcore_barrier()
        base0 = wid * epw
        zero16 = jnp.zeros((16,), F32)
        one16 = jnp.ones((16,), F32)

        def win(w, carry):
            base = base0 + w * K
            pltpu.sync_copy(row_h.at[pl.ds(base, K)], rowb)
            pltpu.sync_copy(col_h.at[pl.ds(base, K)], colb)
            pltpu.sync_copy(ew_h.at[pl.ds(base, K)], ewb)

            def inner(i, c2):
                ix = pl.ds(i * 16, 16)
                rv = rowb[ix]
                cv = colb[ix]
                ev = ewb[ix]
                eq = rv == cv
                ub_deg[ix] = jnp.where(eq, zero16, ev)
                ub_s[ix] = jnp.where(eq, ev, zero16)
                ub_c[ix] = jnp.where(eq, one16, zero16)
                return c2

            lax.fori_loop(0, K // 16, inner, 0)
            pltpu.sync_copy(ub_deg, deg_s.at[colb], add=True)
            pltpu.sync_copy(ub_s, s_s.at[rowb], add=True)
            pltpu.sync_copy(ub_c, c_s.at[rowb], add=True)
            return carry

        lax.fori_loop(0, nwin, win, 0)
        plsc.subcore_barrier()
        pltpu.sync_copy(deg_s.at[sl], degp_h.at[cid, sl])
        pltpu.sync_copy(s_s.at[sl], sp_h.at[cid, sl])
        pltpu.sync_copy(c_s.at[sl], cp_h.at[cid, sl])

    return pl.kernel(
        body,
        out_type=[jax.ShapeDtypeStruct((NC, n), F32)] * 3,
        mesh=_mesh(),
        compiler_params=_SC_PARAMS,
        scratch_types=[
            pltpu.VMEM_SHARED((n,), F32),
            pltpu.VMEM_SHARED((n,), F32),
            pltpu.VMEM_SHARED((n,), F32),
            pltpu.VMEM((K,), I32),
            pltpu.VMEM((K,), I32),
            pltpu.VMEM((K,), F32),
            pltpu.VMEM((K,), F32),
            pltpu.VMEM((K,), F32),
            pltpu.VMEM((K,), F32),
        ],
    )


def _sc_norm(n, e):
    epw = e // NW
    K = 2048
    nwin = epw // K

    def body(row_h, col_h, ew_h, dis_h, norm_h,
             disv, rowb, colb, ewb, nrmb):
        cid = lax.axis_index("c")
        sid = lax.axis_index("s")
        wid = cid * NS + sid
        pltpu.sync_copy(dis_h, disv)
        base0 = wid * epw
        zero16 = jnp.zeros((16,), F32)

        def win(w, carry):
            base = base0 + w * K
            pltpu.sync_copy(row_h.at[pl.ds(base, K)], rowb)
            pltpu.sync_copy(col_h.at[pl.ds(base, K)], colb)
            pltpu.sync_copy(ew_h.at[pl.ds(base, K)], ewb)

            def inner(i, c2):
                ix = pl.ds(i * 16, 16)
                rv = rowb[ix]
                cv = colb[ix]
                ev = ewb[ix]
                dr = plsc.load_gather(disv, [rv])
                dc = plsc.load_gather(disv, [cv])
                ew0 = jnp.where(rv == cv, zero16, ev)
                nrmb[ix] = dr * ew0 * dc
                return c2

            lax.fori_loop(0, K // 16, inner, 0)
            pltpu.sync_copy(nrmb, norm_h.at[pl.ds(base, K)])
            return carry

        lax.fori_loop(0, nwin, win, 0)

    return pl.kernel(
        body,
        out_type=jax.ShapeDtypeStruct((e,), F32),
        mesh=_mesh(),
        compiler_params=_SC_PARAMS,
        scratch_types=[
            pltpu.VMEM((n,), F32),
            pltpu.VMEM((K,), I32),
            pltpu.VMEM((K,), I32),
            pltpu.VMEM((K,), F32),
            pltpu.VMEM((K,), F32),
        ],
    )


def _sc_layer(n, e, hd):
    epw = e // NW
    K = 1024
    nwin = epw // K
    nout = (nwin + 3) // 4
    npc = n // NS

    def body(h_h, row_h, col_h, nrm_h, zero_h, out_h,
             h_s, o_s,
             rb0, rb1, rb2, rb3, cb0, cb1, cb2, cb3,
             nb0, nb1, nb2, nb3, r2a, r2b,
             si0, si1, si2, si3, sg0, sg1, ss0, ss1):
        rbs = [rb0, rb1, rb2, rb3]
        cbs = [cb0, cb1, cb2, cb3]
        nbs = [nb0, nb1, nb2, nb3]
        rows2 = [r2a, r2b]
        upd2 = rows2
        sis = [si0, si1, si2, si3]
        sgs = [sg0, sg1]
        sss = [ss0, ss1]

        cid = lax.axis_index("c")
        sid = lax.axis_index("s")
        wid = cid * NS + sid
        sl = pl.ds(sid * npc, npc)
        base0 = wid * epw
        iota = lax.iota(I32, 16)
        lane8 = lax.shift_right_logical(iota, 3)
        lane7 = lax.bitwise_and(iota, jnp.full((16,), 7, I32))

        def issue_streams(w, s4):
            base = base0 + w * K
            pltpu.async_copy(row_h.at[pl.ds(base, K)], rbs[s4], sis[s4])
            pltpu.async_copy(col_h.at[pl.ds(base, K)], cbs[s4], sis[s4])
            pltpu.async_copy(nrm_h.at[pl.ds(base, K)], nbs[s4], sis[s4])

        def wait_streams(s4):
            pltpu.make_async_copy(row_h.at[pl.ds(0, K)], rbs[s4], sis[s4]).wait()
            pltpu.make_async_copy(col_h.at[pl.ds(0, K)], cbs[s4], sis[s4]).wait()
            pltpu.make_async_copy(nrm_h.at[pl.ds(0, K)], nbs[s4], sis[s4]).wait()

        def issue_gather(s4, s2):
            pltpu.async_copy(h_s.at[rbs[s4]], rows2[s2], sgs[s2])

        def wait_gather(s4, s2):
            pltpu.make_async_copy(h_s.at[rbs[s4]], rows2[s2], sgs[s2]).wait()

        def issue_scatter(s4, s2):
            pltpu.async_copy(upd2[s2], o_s.at[cbs[s4]], sss[s2], add=True)

        def wait_scatter(s4, s2):
            pltpu.make_async_copy(upd2[s2], o_s.at[cbs[s4]], sss[s2]).wait()

        issue_streams(0, 0)
        issue_streams(1, 1)
        pltpu.sync_copy(h_h.at[sl], h_s.at[sl])
        pltpu.sync_copy(zero_h.at[sl], o_s.at[sl])
        plsc.subcore_barrier()
        wait_streams(0)
        issue_gather(0, 0)

        def outer(it, carry):
            w0 = it * 4
            for j in range(4):
                w = w0 + j
                s2 = j % 2

                @pl.when(w < nwin)
                def _window():
                    wait_gather(j, s2)

                    @pl.when(w + 2 < nwin)
                    def _():
                        issue_streams(w + 2, (j + 2) % 4)

                    @pl.when(w + 1 < nwin)
                    def _():
                        wait_streams((j + 1) % 4)

                        @pl.when(w >= 1)
                        def _():
                            wait_scatter((j + 3) % 4, (j + 1) % 2)

                        issue_gather((j + 1) % 4, (j + 1) % 2)

                    r2f = rows2[s2]

                    def inner(i):
                        e_idx = lane8 + i * 2
                        v = plsc.load_gather(r2f, [e_idx, lane7])
                        nv = plsc.load_gather(nbs[j], [e_idx])
                        plsc.store_scatter(r2f, [e_idx, lane7], v * nv)

                    plsc.parallel_loop(0, K // 2, 1, unroll=8)(inner)
                    issue_scatter(j, s2)

            return carry

        lax.fori_loop(0, nout, outer, 0)
        wait_scatter((nwin - 2) % 4, (nwin - 2) % 2)
        wait_scatter((nwin - 1) % 4, (nwin - 1) % 2)
        plsc.subcore_barrier()
        pltpu.sync_copy(o_s.at[sl], out_h.at[cid, sl])

    return pl.kernel(
        body,
        out_type=jax.ShapeDtypeStruct((NC, n, hd), F32),
        mesh=_mesh(),
        compiler_params=_SC_PARAMS,
        scratch_types=(
            [pltpu.VMEM_SHARED((n, hd), F32),
             pltpu.VMEM_SHARED((n, hd), F32)]
            + [pltpu.VMEM((K,), I32)] * 8
            + [pltpu.VMEM((K,), F32)] * 4
            + [pltpu.VMEM((K, hd), F32)] * 2
            + [pltpu.SemaphoreType.DMA] * 8
        ),
    )


def _tc_matmul(n, in_dim, hd):
    P = 128 // hd
    Bp = 512

    def body(x_ref, m_ref, o_ref):
        acc = jnp.zeros((Bp, 128), F32)
        for v in range(P):
            acc = acc + jnp.dot(x_ref[:, v, :], m_ref[v],
                                preferred_element_type=F32)
        o_ref[...] = acc

    return pl.pallas_call(
        body,
        grid=(n // P // Bp,),
        in_specs=[pl.BlockSpec((Bp, P, in_dim), lambda i: (i, 0, 0)),
                  pl.BlockSpec((P, in_dim, 128), lambda i: (0, 0, 0))],
        out_specs=pl.BlockSpec((Bp, 128), lambda i: (i, 0)),
        out_shape=jax.ShapeDtypeStruct((n // P, 128), F32),
    )


def _tc_prep(n, hd):
    r = n // 128

    def body(degp, sp, cp, dis_o, d_o):
        raw = degp[0] + degp[1]
        s = sp[0] + sp[1]
        c = cp[0] + cp[1]
        loop_w = jnp.where(c > 0, s / jnp.maximum(c, 1.0),
                           jnp.ones_like(c))
        deg = raw + loop_w
        dis = jnp.where(deg > 0, lax.rsqrt(jnp.where(deg > 0, deg, 1.0)),
                        jnp.zeros_like(deg))
        dis_o[...] = dis
        d_o[...] = dis * dis * loop_w

    return pl.pallas_call(
        body,
        out_shape=[jax.ShapeDtypeStruct((r, 128), F32),
                   jax.ShapeDtypeStruct((r, 128), F32)],
    )


def _tc_comb(n, hd, with_w, w2_dim):
    P = 128 // hd
    npk = n // P
    Bp = 512

    def body(*refs):
        if w2_dim:
            p_ref, d_ref, h_ref, w_ref, w2_ref, o_ref = refs
        elif with_w:
            p_ref, d_ref, h_ref, w_ref, o_ref = refs
        else:
            p_ref, d_ref, h_ref, o_ref = refs
        agg = p_ref[0] + p_ref[1] + d_ref[...] * h_ref[...]
        if with_w:
            agg = jnp.dot(agg, w_ref[...], preferred_element_type=F32)
        z = jnp.maximum(agg, 0.0)
        if w2_dim:
            z = jnp.dot(z, w2_ref[...], preferred_element_type=F32)
        o_ref[...] = z

    in_specs = [pl.BlockSpec((NC, Bp, 128), lambda i: (0, i, 0)),
                pl.BlockSpec((Bp, 128), lambda i: (i, 0)),
                pl.BlockSpec((Bp, 128), lambda i: (i, 0))]
    if with_w:
        in_specs.append(pl.BlockSpec((128, 128), lambda i: (0, 0)))
    if w2_dim:
        odim = P * w2_dim
        in_specs.append(pl.BlockSpec((128, odim), lambda i: (0, 0)))
        out_spec = pl.BlockSpec((Bp, odim), lambda i: (i, 0))
        out_shape = jax.ShapeDtypeStruct((npk, odim), F32)
    else:
        out_spec = pl.BlockSpec((Bp, 128), lambda i: (i, 0))
        out_shape = jax.ShapeDtypeStruct((npk, 128), F32)

    return pl.pallas_call(
        body,
        grid=(npk // Bp,),
        in_specs=in_specs,
        out_specs=out_spec,
        out_shape=out_shape,
    )


def kernel(x, edge_index, edge_weight, W0, W1, W2, Wlin):
    n, in_dim = x.shape
    e = edge_index.shape[1]
    hd = W0.shape[1]
    row = edge_index[0]
    col = edge_index[1]
    zeros1 = jnp.zeros((n,), F32)
    zeros2 = jnp.zeros((n, hd), F32)

    P = 128 // hd
    npk = n // P
    r = n // 128

    m0 = jnp.stack([jnp.pad(W0, ((0, 0), (v * hd, 128 - (v + 1) * hd)))
                    for v in range(P)])
    h0_pk = _tc_matmul(n, in_dim, hd)(x.reshape(npk, P, in_dim), m0)

    degp, sp, cp = _sc_deg(n, e)(row, col, edge_weight, zeros1)
    dis2d, d2d = _tc_prep(n, hd)(degp.reshape(NC, r, 128),
                                 sp.reshape(NC, r, 128),
                                 cp.reshape(NC, r, 128))
    dis = dis2d.reshape(n)
    d_pk = jnp.repeat(d2d.reshape(n), hd).reshape(npk, 128)

    nrm = _sc_norm(n, e)(row, col, edge_weight, dis)

    eye_p = jnp.eye(P, dtype=F32)
    w1bd = jnp.kron(eye_p, W1)
    w2bd = jnp.kron(eye_p, W2)
    wlbd = jnp.kron(eye_p, Wlin)

    layer = _sc_layer(n, e, hd)
    comb_relu = _tc_comb(n, hd, False, 0)
    comb_w = _tc_comb(n, hd, True, 0)

    p = layer(h0_pk.reshape(n, hd), row, col, nrm, zeros2)
    h1_pk = comb_relu(p.reshape(NC, npk, 128), d_pk, h0_pk)
    p = layer(h1_pk.reshape(n, hd), row, col, nrm, zeros2)
    h2_pk = comb_w(p.reshape(NC, npk, 128), d_pk, h1_pk, w1bd)
    p = layer(h2_pk.reshape(n, hd), row, col, nrm, zeros2)
    pred_pk = _tc_comb(n, hd, True, Wlin.shape[1])(p.reshape(NC, npk, 128),
                                                   d_pk, h2_pk, w2bd, wlbd)

    g = 6
    return pred_pk.reshape(n)[::g].reshape(n // g, 1)


# speedup vs baseline: 126.2190x; 1.1544x over previous
import functools

import jax
import jax.numpy as jnp
from jax import lax
from jax.experimental import pallas as pl
from jax.experimental.pallas import tpu as pltpu
from jax.experimental.pallas import tpu_sc as plsc

NC = 2
NS = 16
NW = NC * NS

F32 = jnp.float32
I32 = jnp.int32


def _mesh():
    return plsc.VectorSubcoreMesh(core_axis_name="c", subcore_axis_name="s")


_SC_PARAMS = pltpu.CompilerParams(needs_layout_passes=False,
                                  use_tc_tiling_on_sc=False)


def _sc_deg(n, e):
    epw = e // NW
    K = 2048
    nwin = epw // K
    nout = (nwin + 3) // 4
    npc = n // NS

    def body(row_h, col_h, ew_h, zero_h, degp_h, cbp_h,
             deg_s, cmb_s,
             rb0, rb1, rb2, rb3, cb0, cb1, cb2, cb3,
             eb0, eb1, eb2, eb3, ud0, ud1, uc0, uc1,
             si0, si1, si2, si3, sd0, sd1, sc0, sc1):
        rbs = [rb0, rb1, rb2, rb3]
        cbs = [cb0, cb1, cb2, cb3]
        ebs = [eb0, eb1, eb2, eb3]
        uds = [ud0, ud1]
        ucs = [uc0, uc1]
        sis = [si0, si1, si2, si3]
        sds = [sd0, sd1]
        scs = [sc0, sc1]

        cid = lax.axis_index("c")
        sid = lax.axis_index("s")
        wid = cid * NS + sid
        sl = pl.ds(sid * npc, npc)
        base0 = wid * epw
        zero16 = jnp.zeros((16,), F32)
        eight16 = jnp.full((16,), 8.0, F32)

        def issue_in(w, s4):
            base = base0 + w * K
            pltpu.async_copy(row_h.at[pl.ds(base, K)], rbs[s4], sis[s4])
            pltpu.async_copy(col_h.at[pl.ds(base, K)], cbs[s4], sis[s4])
            pltpu.async_copy(ew_h.at[pl.ds(base, K)], ebs[s4], sis[s4])

        def wait_in(s4):
            pltpu.make_async_copy(row_h.at[pl.ds(0, K)], rbs[s4], sis[s4]).wait()
            pltpu.make_async_copy(col_h.at[pl.ds(0, K)], cbs[s4], sis[s4]).wait()
            pltpu.make_async_copy(ew_h.at[pl.ds(0, K)], ebs[s4], sis[s4]).wait()

        def issue_scat(s4, s2):
            pltpu.async_copy(uds[s2], deg_s.at[cbs[s4]], sds[s2], add=True)
            pltpu.async_copy(ucs[s2], cmb_s.at[rbs[s4]], scs[s2], add=True)

        def wait_scat(s4, s2):
            pltpu.make_async_copy(uds[s2], deg_s.at[cbs[s4]], sds[s2]).wait()
            pltpu.make_async_copy(ucs[s2], cmb_s.at[rbs[s4]], scs[s2]).wait()

        issue_in(0, 0)
        issue_in(1, 1)
        pltpu.sync_copy(zero_h.at[sl], deg_s.at[sl])
        pltpu.sync_copy(zero_h.at[sl], cmb_s.at[sl])
        plsc.subcore_barrier()

        def outer(it, carry):
            w0 = it * 4
            for j in range(4):
                w = w0 + j
                s2 = j % 2

                @pl.when(w < nwin)
                def _window():
                    wait_in(j)

                    @pl.when(w >= 2)
                    def _():
                        wait_scat((j + 2) % 4, s2)

                    def inner(i):
                        ix = pl.ds(i * 16, 16)
                        rv = rbs[j][ix]
                        cv = cbs[j][ix]
                        ev = ebs[j][ix]
                        eq = rv == cv
                        uds[s2][ix] = jnp.where(eq, zero16, ev)
                        ucs[s2][ix] = jnp.where(eq, ev + eight16, zero16)

                    plsc.parallel_loop(0, K // 16, 1, unroll=8)(inner)
                    issue_scat(j, s2)

                    @pl.when(w + 2 < nwin)
                    def _():
                        issue_in(w + 2, (j + 2) % 4)

            return carry

        lax.fori_loop(0, nout, outer, 0)
        wait_scat((nwin - 2) % 4, (nwin - 2) % 2)
        wait_scat((nwin - 1) % 4, (nwin - 1) % 2)
        plsc.subcore_barrier()
        pltpu.sync_copy(deg_s.at[sl], degp_h.at[cid, sl])
        pltpu.sync_copy(cmb_s.at[sl], cbp_h.at[cid, sl])

    return pl.kernel(
        body,
        out_type=[jax.ShapeDtypeStruct((NC, n), F32)] * 2,
        mesh=_mesh(),
        compiler_params=_SC_PARAMS,
        scratch_types=(
            [pltpu.VMEM_SHARED((n,), F32),
             pltpu.VMEM_SHARED((n,), F32)]
            + [pltpu.VMEM((K,), I32)] * 8
            + [pltpu.VMEM((K,), F32)] * 4
            + [pltpu.VMEM((K,), F32)] * 4
            + [pltpu.SemaphoreType.DMA] * 8
        ),
    )


def _sc_norm(n, e):
    epw = e // NW
    K = 2048
    nwin = epw // K

    nout = (nwin + 3) // 4

    def body(row_h, col_h, ew_h, dis_h, norm_h,
             disv,
             rb0, rb1, rb2, rb3, cb0, cb1, cb2, cb3,
             eb0, eb1, eb2, eb3, nb0, nb1,
             si0, si1, si2, si3, so0, so1):
        rbs = [rb0, rb1, rb2, rb3]
        cbs = [cb0, cb1, cb2, cb3]
        ebs = [eb0, eb1, eb2, eb3]
        nbs = [nb0, nb1]
        sis = [si0, si1, si2, si3]
        sos = [so0, so1]

        cid = lax.axis_index("c")
        sid = lax.axis_index("s")
        wid = cid * NS + sid
        base0 = wid * epw
        zero16 = jnp.zeros((16,), F32)

        def issue_in(w, s4):
            base = base0 + w * K
            pltpu.async_copy(row_h.at[pl.ds(base, K)], rbs[s4], sis[s4])
            pltpu.async_copy(col_h.at[pl.ds(base, K)], cbs[s4], sis[s4])
            pltpu.async_copy(ew_h.at[pl.ds(base, K)], ebs[s4], sis[s4])

        def wait_in(s4):
            pltpu.make_async_copy(row_h.at[pl.ds(0, K)], rbs[s4], sis[s4]).wait()
            pltpu.make_async_copy(col_h.at[pl.ds(0, K)], cbs[s4], sis[s4]).wait()
            pltpu.make_async_copy(ew_h.at[pl.ds(0, K)], ebs[s4], sis[s4]).wait()

        def issue_out(w, s2):
            pltpu.async_copy(nbs[s2], norm_h.at[pl.ds(base0 + w * K, K)],
                             sos[s2])

        def wait_out(s2):
            pltpu.make_async_copy(nbs[s2], norm_h.at[pl.ds(0, K)],
                                  sos[s2]).wait()

        issue_in(0, 0)
        issue_in(1, 1)
        pltpu.sync_copy(dis_h, disv)

        def outer(it, carry):
            w0 = it * 4
            for j in range(4):
                w = w0 + j
                s2 = j % 2

                @pl.when(w < nwin)
                def _window():
                    wait_in(j)

                    @pl.when(w >= 2)
                    def _():
                        wait_out(s2)

                    def inner(i):
                        ix = pl.ds(i * 16, 16)
                        rv = rbs[j][ix]
                        cv = cbs[j][ix]
                        ev = ebs[j][ix]
                        dr = plsc.load_gather(disv, [rv])
                        dc = plsc.load_gather(disv, [cv])
                        ew0 = jnp.where(rv == cv, zero16, ev)
                        nbs[s2][ix] = dr * ew0 * dc

                    plsc.parallel_loop(0, K // 16, 1, unroll=8)(inner)
                    issue_out(w, s2)

                    @pl.when(w + 2 < nwin)
                    def _():
                        issue_in(w + 2, (j + 2) % 4)

            return carry

        lax.fori_loop(0, nout, outer, 0)
        wait_out((nwin - 2) % 2)
        wait_out((nwin - 1) % 2)

    return pl.kernel(
        body,
        out_type=jax.ShapeDtypeStruct((e,), F32),
        mesh=_mesh(),
        compiler_params=_SC_PARAMS,
        scratch_types=(
            [pltpu.VMEM((n,), F32)]
            + [pltpu.VMEM((K,), I32)] * 8
            + [pltpu.VMEM((K,), F32)] * 4
            + [pltpu.VMEM((K,), F32)] * 2
            + [pltpu.SemaphoreType.DMA] * 6
        ),
    )


def _sc_layer(n, e, hd):
    epw = e // NW
    K = 1024
    nwin = epw // K
    nout = (nwin + 3) // 4
    npc = n // NS

    def body(h_h, row_h, col_h, nrm_h, zero_h, out_h,
             h_s, o_s,
             rb0, rb1, rb2, rb3, cb0, cb1, cb2, cb3,
             nb0, nb1, nb2, nb3, r2a, r2b,
             si0, si1, si2, si3, sg0, sg1, ss0, ss1):
        rbs = [rb0, rb1, rb2, rb3]
        cbs = [cb0, cb1, cb2, cb3]
        nbs = [nb0, nb1, nb2, nb3]
        rows2 = [r2a, r2b]
        upd2 = rows2
        sis = [si0, si1, si2, si3]
        sgs = [sg0, sg1]
        sss = [ss0, ss1]

        cid = lax.axis_index("c")
        sid = lax.axis_index("s")
        wid = cid * NS + sid
        sl = pl.ds(sid * npc, npc)
        base0 = wid * epw
        iota = lax.iota(I32, 16)
        lane8 = lax.shift_right_logical(iota, 3)
        lane7 = lax.bitwise_and(iota, jnp.full((16,), 7, I32))

        def issue_streams(w, s4):
            base = base0 + w * K
            pltpu.async_copy(row_h.at[pl.ds(base, K)], rbs[s4], sis[s4])
            pltpu.async_copy(col_h.at[pl.ds(base, K)], cbs[s4], sis[s4])
            pltpu.async_copy(nrm_h.at[pl.ds(base, K)], nbs[s4], sis[s4])

        def wait_streams(s4):
            pltpu.make_async_copy(row_h.at[pl.ds(0, K)], rbs[s4], sis[s4]).wait()
            pltpu.make_async_copy(col_h.at[pl.ds(0, K)], cbs[s4], sis[s4]).wait()
            pltpu.make_async_copy(nrm_h.at[pl.ds(0, K)], nbs[s4], sis[s4]).wait()

        def issue_gather(s4, s2):
            pltpu.async_copy(h_s.at[rbs[s4]], rows2[s2], sgs[s2])

        def wait_gather(s4, s2):
            pltpu.make_async_copy(h_s.at[rbs[s4]], rows2[s2], sgs[s2]).wait()

        def issue_scatter(s4, s2):
            pltpu.async_copy(upd2[s2], o_s.at[cbs[s4]], sss[s2], add=True)

        def wait_scatter(s4, s2):
            pltpu.make_async_copy(upd2[s2], o_s.at[cbs[s4]], sss[s2]).wait()

        issue_streams(0, 0)
        issue_streams(1, 1)
        pltpu.sync_copy(h_h.at[sl], h_s.at[sl])
        pltpu.sync_copy(zero_h.at[sl], o_s.at[sl])
        plsc.subcore_barrier()
        wait_streams(0)
        issue_gather(0, 0)

        def outer(it, carry):
            w0 = it * 4
            for j in range(4):
                w = w0 + j
                s2 = j % 2

                @pl.when(w < nwin)
                def _window():
                    wait_gather(j, s2)

                    @pl.when(w + 2 < nwin)
                    def _():
                        issue_streams(w + 2, (j + 2) % 4)

                    @pl.when(w + 1 < nwin)
                    def _():
                        wait_streams((j + 1) % 4)

                        @pl.when(w >= 1)
                        def _():
                            wait_scatter((j + 3) % 4, (j + 1) % 2)

                        issue_gather((j + 1) % 4, (j + 1) % 2)

                    r2f = rows2[s2]

                    def inner(i):
                        e_idx = lane8 + i * 2
                        v = plsc.load_gather(r2f, [e_idx, lane7])
                        nv = plsc.load_gather(nbs[j], [e_idx])
                        plsc.store_scatter(r2f, [e_idx, lane7], v * nv)

                    plsc.parallel_loop(0, K // 2, 1, unroll=8)(inner)
                    issue_scatter(j, s2)

            return carry

        lax.fori_loop(0, nout, outer, 0)
        wait_scatter((nwin - 2) % 4, (nwin - 2) % 2)
        wait_scatter((nwin - 1) % 4, (nwin - 1) % 2)
        plsc.subcore_barrier()
        pltpu.sync_copy(o_s.at[sl], out_h.at[cid, sl])

    return pl.kernel(
        body,
        out_type=jax.ShapeDtypeStruct((NC, n, hd), F32),
        mesh=_mesh(),
        compiler_params=_SC_PARAMS,
        scratch_types=(
            [pltpu.VMEM_SHARED((n, hd), F32),
             pltpu.VMEM_SHARED((n, hd), F32)]
            + [pltpu.VMEM((K,), I32)] * 8
            + [pltpu.VMEM((K,), F32)] * 4
            + [pltpu.VMEM((K, hd), F32)] * 2
            + [pltpu.SemaphoreType.DMA] * 8
        ),
    )


def _tc_matmul(n, in_dim, hd):
    P = 128 // hd
    Bp = 512

    def body(x_ref, m_ref, o_ref):
        acc = jnp.zeros((Bp, 128), F32)
        for v in range(P):
            acc = acc + jnp.dot(x_ref[:, v, :], m_ref[v],
                                preferred_element_type=F32)
        o_ref[...] = acc

    return pl.pallas_call(
        body,
        grid=(n // P // Bp,),
        in_specs=[pl.BlockSpec((Bp, P, in_dim), lambda i: (i, 0, 0)),
                  pl.BlockSpec((P, in_dim, 128), lambda i: (0, 0, 0))],
        out_specs=pl.BlockSpec((Bp, 128), lambda i: (i, 0)),
        out_shape=jax.ShapeDtypeStruct((n // P, 128), F32),
    )


def _tc_prep(n, hd):
    r = n // 128

    def body(degp, cbp, dis_o, d_o):
        raw = degp[0] + degp[1]
        combo = cbp[0] + cbp[1]
        c = jnp.floor(combo * 0.125)
        s = combo - 8.0 * c
        loop_w = jnp.where(c > 0, s / jnp.maximum(c, 1.0),
                           jnp.ones_like(c))
        deg = raw + loop_w
        dis = jnp.where(deg > 0, lax.rsqrt(jnp.where(deg > 0, deg, 1.0)),
                        jnp.zeros_like(deg))
        dis_o[...] = dis
        d_o[...] = dis * dis * loop_w

    return pl.pallas_call(
        body,
        out_shape=[jax.ShapeDtypeStruct((r, 128), F32),
                   jax.ShapeDtypeStruct((r, 128), F32)],
    )


def _tc_comb(n, hd, with_w, w2_dim):
    P = 128 // hd
    npk = n // P
    Bp = 512

    def body(*refs):
        if w2_dim:
            p_ref, d_ref, h_ref, w_ref, w2_ref, o_ref = refs
        elif with_w:
            p_ref, d_ref, h_ref, w_ref, o_ref = refs
        else:
            p_ref, d_ref, h_ref, o_ref = refs
        agg = p_ref[0] + p_ref[1] + d_ref[...] * h_ref[...]
        if with_w:
            agg = jnp.dot(agg, w_ref[...], preferred_element_type=F32)
        z = jnp.maximum(agg, 0.0)
        if w2_dim:
            z = jnp.dot(z, w2_ref[...], preferred_element_type=F32)
        o_ref[...] = z

    in_specs = [pl.BlockSpec((NC, Bp, 128), lambda i: (0, i, 0)),
                pl.BlockSpec((Bp, 128), lambda i: (i, 0)),
                pl.BlockSpec((Bp, 128), lambda i: (i, 0))]
    if with_w:
        in_specs.append(pl.BlockSpec((128, 128), lambda i: (0, 0)))
    if w2_dim:
        odim = P * w2_dim
        in_specs.append(pl.BlockSpec((128, odim), lambda i: (0, 0)))
        out_spec = pl.BlockSpec((Bp, odim), lambda i: (i, 0))
        out_shape = jax.ShapeDtypeStruct((npk, odim), F32)
    else:
        out_spec = pl.BlockSpec((Bp, 128), lambda i: (i, 0))
        out_shape = jax.ShapeDtypeStruct((npk, 128), F32)

    return pl.pallas_call(
        body,
        grid=(npk // Bp,),
        in_specs=in_specs,
        out_specs=out_spec,
        out_shape=out_shape,
    )


def kernel(x, edge_index, edge_weight, W0, W1, W2, Wlin):
    n, in_dim = x.shape
    e = edge_index.shape[1]
    hd = W0.shape[1]
    row = edge_index[0]
    col = edge_index[1]
    zeros1 = jnp.zeros((n,), F32)
    zeros2 = jnp.zeros((n, hd), F32)

    P = 128 // hd
    npk = n // P
    r = n // 128

    m0 = jnp.stack([jnp.pad(W0, ((0, 0), (v * hd, 128 - (v + 1) * hd)))
                    for v in range(P)])
    h0_pk = _tc_matmul(n, in_dim, hd)(x.reshape(npk, P, in_dim), m0)

    degp, cbp = _sc_deg(n, e)(row, col, edge_weight, zeros1)
    dis2d, d2d = _tc_prep(n, hd)(degp.reshape(NC, r, 128),
                                 cbp.reshape(NC, r, 128))
    dis = dis2d.reshape(n)
    d_pk = jnp.repeat(d2d.reshape(n), hd).reshape(npk, 128)

    nrm = _sc_norm(n, e)(row, col, edge_weight, dis)

    eye_p = jnp.eye(P, dtype=F32)
    w1bd = jnp.kron(eye_p, W1)
    w2bd = jnp.kron(eye_p, W2)
    wlbd = jnp.kron(eye_p, Wlin)

    layer = _sc_layer(n, e, hd)
    comb_relu = _tc_comb(n, hd, False, 0)
    comb_w = _tc_comb(n, hd, True, 0)

    p = layer(h0_pk.reshape(n, hd), row, col, nrm, zeros2)
    h1_pk = comb_relu(p.reshape(NC, npk, 128), d_pk, h0_pk)
    p = layer(h1_pk.reshape(n, hd), row, col, nrm, zeros2)
    h2_pk = comb_w(p.reshape(NC, npk, 128), d_pk, h1_pk, w1bd)
    p = layer(h2_pk.reshape(n, hd), row, col, nrm, zeros2)
    pred_pk = _tc_comb(n, hd, True, Wlin.shape[1])(p.reshape(NC, npk, 128),
                                                   d_pk, h2_pk, w2bd, wlbd)

    g = 6
    return pred_pk.reshape(n)[::g].reshape(n // g, 1)
